# baseline (speedup 1.0000x reference)
import functools

import jax
import jax.numpy as jnp
from jax import lax
from jax.experimental import pallas as pl
from jax.experimental.pallas import tpu as pltpu, tpu_sc as plsc

N = 32768
D = 256
K = 8192
RB = 256
NRB = N // RB
CW = 128
LW = 128

NC = 2
NS = 16
NW = NC * NS
BPW = N // NW
CH = 128
NCHUNK = BPW // CH


def _argmin_body(x_ref, v_ref, aux_ref, idx_ref, loss_ref, acc_ref):
    i = pl.program_id(0)
    x = x_ref[...]
    x2 = jnp.sum(x * x, axis=1, keepdims=True)
    xs = x * -2.0

    amin = jnp.full((RB, LW), jnp.float32(jnp.inf))
    aarg = jnp.full((RB, LW), jnp.float32(K))
    for c in range(K // CW):
        sl = slice(c * CW, (c + 1) * CW)
        xvc = jnp.dot(xs, v_ref[:, sl], preferred_element_type=jnp.float32)
        sc = (x2 + xvc) + aux_ref[0:1, sl]
        for l in range(CW // LW):
            s_l = sc[:, l * LW:(l + 1) * LW]
            col_l = jnp.broadcast_to(
                aux_ref[1:2, c * CW + l * LW: c * CW + (l + 1) * LW], (RB, LW)
            )
            aarg = jnp.where(s_l < amin, col_l, aarg)
            amin = jnp.minimum(s_l, amin)
    rowmin = jnp.min(amin, axis=1, keepdims=True)
    masked = jnp.where(amin == rowmin, aarg, jnp.float32(K))
    idx_ref[...] = jnp.min(masked, axis=1, keepdims=True).astype(jnp.int32)

    part = jnp.sum(rowmin, axis=0, keepdims=True)

    @pl.when(i == 0)
    def _():
        acc_ref[...] = part

    @pl.when(i > 0)
    def _():
        acc_ref[...] = acc_ref[...] + part

    @pl.when(i == NRB - 1)
    def _():
        loss_ref[...] = acc_ref[...]


_argmin_call = pl.pallas_call(
    _argmin_body,
    grid=(NRB,),
    in_specs=[
        pl.BlockSpec((RB, D), lambda i: (i, 0)),
        pl.BlockSpec((D, K), lambda i: (0, 0)),
        pl.BlockSpec((2, K), lambda i: (0, 0)),
    ],
    out_specs=[
        pl.BlockSpec((RB, 1), lambda i: (i, 0)),
        pl.BlockSpec((1, 1), lambda i: (0, 0)),
    ],
    out_shape=[
        jax.ShapeDtypeStruct((N, 1), jnp.int32),
        jax.ShapeDtypeStruct((1, 1), jnp.float32),
    ],
    scratch_shapes=[
        pltpu.VMEM((1, 1), jnp.float32),
    ],
    compiler_params=pltpu.CompilerParams(
        dimension_semantics=("arbitrary",),
    ),
)


@functools.cache
def _make_gather():
    @functools.partial(
        pl.kernel,
        mesh=plsc.VectorSubcoreMesh(
            core_axis_name="c", subcore_axis_name="s", num_cores=NC
        ),
        out_type=jax.ShapeDtypeStruct((N, D), jnp.float32),
        scratch_types=[
            pltpu.VMEM((2, CH), jnp.int32),
            pltpu.VMEM((CH, D), jnp.float32),
            pltpu.VMEM((CH, D), jnp.float32),
            pltpu.SemaphoreType.DMA,
            pltpu.SemaphoreType.DMA,
            pltpu.SemaphoreType.DMA,
            pltpu.SemaphoreType.DMA,
        ],
    )
    def _gather(table_hbm, idx_hbm, out_hbm, idx_v, rows_v0, rows_v1,
                sem_g0, sem_g1, sem_w0, sem_w1):
        wid = lax.axis_index("s") * NC + lax.axis_index("c")
        base = wid * BPW
        rows = (rows_v0, rows_v1)
        sem_g = (sem_g0, sem_g1)
        sem_w = (sem_w0, sem_w1)

        pltpu.sync_copy(idx_hbm.at[pl.ds(base, CH)], idx_v.at[0])
        g = pltpu.async_copy(table_hbm.at[idx_v.at[0]], rows_v0, sem_g0)
        for c in range(1, NCHUNK):
            b = c % 2
            pltpu.sync_copy(
                idx_hbm.at[pl.ds(base + c * CH, CH)], idx_v.at[b]
            )
            if c >= 2:
                pltpu.make_async_copy(
                    rows[b], out_hbm.at[pl.ds(base + (c - 2) * CH, CH)],
                    sem_w[b],
                ).wait()
            g_next = pltpu.async_copy(
                table_hbm.at[idx_v.at[b]], rows[b], sem_g[b]
            )
            g.wait()
            pltpu.async_copy(
                rows[1 - b], out_hbm.at[pl.ds(base + (c - 1) * CH, CH)],
                sem_w[1 - b],
            )
            g = g_next
        last = NCHUNK - 1
        b = last % 2
        g.wait()
        pltpu.async_copy(
            rows[b], out_hbm.at[pl.ds(base + last * CH, CH)], sem_w[b]
        ).wait()
        if NCHUNK >= 2:
            pltpu.make_async_copy(
                rows[1 - b], out_hbm.at[pl.ds(base + (last - 1) * CH, CH)],
                sem_w[1 - b],
            ).wait()

    return _gather


def kernel(x, vectors):
    aux = jnp.concatenate(
        [
            jnp.sum(vectors * vectors, axis=0, keepdims=True),
            jnp.arange(K, dtype=jnp.float32)[None, :],
        ],
        axis=0,
    )
    idx2d, loss_sum = _argmin_call(x, vectors, aux)
    quantized = _make_gather()(vectors.T, idx2d.reshape(-1))
    l = loss_sum[0, 0] / (N * D)
    return quantized, l, l, idx2d


# speedup vs baseline: 1.3321x; 1.3321x over previous
import functools

import jax
import jax.numpy as jnp
from jax import lax
from jax.experimental import pallas as pl
from jax.experimental.pallas import tpu as pltpu, tpu_sc as plsc

N = 32768
D = 256
K = 8192
RB = 512
NRB = N // RB
CW = 256
LW = 128

NC = 2
NS = 16
NW = NC * NS
BPW = N // NW
CH = 128
NCHUNK = BPW // CH


def _argmin_body(x_ref, v_ref, aux_ref, idx_ref, loss_ref, acc_ref):
    i = pl.program_id(0)
    x = x_ref[...]
    x2 = jnp.sum(x * x, axis=1, keepdims=True)
    xs = x * -2.0

    amin = jnp.full((RB, LW), jnp.float32(jnp.inf))
    aarg = jnp.full((RB, LW), jnp.float32(K))
    for c in range(K // CW):
        sl = slice(c * CW, (c + 1) * CW)
        xvc = jnp.dot(xs, v_ref[:, sl], preferred_element_type=jnp.float32)
        sc = (x2 + xvc) + aux_ref[0:1, sl]
        for l in range(CW // LW):
            s_l = sc[:, l * LW:(l + 1) * LW]
            col_l = jnp.broadcast_to(
                aux_ref[1:2, c * CW + l * LW: c * CW + (l + 1) * LW], (RB, LW)
            )
            aarg = jnp.where(s_l < amin, col_l, aarg)
            amin = jnp.minimum(s_l, amin)
    rowmin = jnp.min(amin, axis=1, keepdims=True)
    masked = jnp.where(amin == rowmin, aarg, jnp.float32(K))
    idx_ref[...] = jnp.min(masked, axis=1, keepdims=True).astype(jnp.int32)

    part = jnp.sum(rowmin, axis=0, keepdims=True)

    @pl.when(i == 0)
    def _():
        acc_ref[...] = part

    @pl.when(i > 0)
    def _():
        acc_ref[...] = acc_ref[...] + part

    @pl.when(i == NRB - 1)
    def _():
        loss_ref[...] = acc_ref[...]


_argmin_call = pl.pallas_call(
    _argmin_body,
    grid=(NRB,),
    in_specs=[
        pl.BlockSpec((RB, D), lambda i: (i, 0)),
        pl.BlockSpec((D, K), lambda i: (0, 0)),
        pl.BlockSpec((2, K), lambda i: (0, 0)),
    ],
    out_specs=[
        pl.BlockSpec((RB, 1), lambda i: (i, 0)),
        pl.BlockSpec((1, 1), lambda i: (0, 0)),
    ],
    out_shape=[
        jax.ShapeDtypeStruct((N, 1), jnp.int32),
        jax.ShapeDtypeStruct((1, 1), jnp.float32),
    ],
    scratch_shapes=[
        pltpu.VMEM((1, 1), jnp.float32),
    ],
    compiler_params=pltpu.CompilerParams(
        dimension_semantics=("arbitrary",),
    ),
)


@functools.cache
def _make_gather():
    @functools.partial(
        pl.kernel,
        mesh=plsc.VectorSubcoreMesh(
            core_axis_name="c", subcore_axis_name="s", num_cores=NC
        ),
        out_type=jax.ShapeDtypeStruct((N, D), jnp.float32),
        scratch_types=[
            pltpu.VMEM((2, CH), jnp.int32),
            pltpu.VMEM((CH, D), jnp.float32),
            pltpu.VMEM((CH, D), jnp.float32),
            pltpu.SemaphoreType.DMA,
            pltpu.SemaphoreType.DMA,
            pltpu.SemaphoreType.DMA,
            pltpu.SemaphoreType.DMA,
        ],
    )
    def _gather(table_hbm, idx_hbm, out_hbm, idx_v, rows_v0, rows_v1,
                sem_g0, sem_g1, sem_w0, sem_w1):
        wid = lax.axis_index("s") * NC + lax.axis_index("c")
        base = wid * BPW
        rows = (rows_v0, rows_v1)
        sem_g = (sem_g0, sem_g1)
        sem_w = (sem_w0, sem_w1)

        pltpu.sync_copy(idx_hbm.at[pl.ds(base, CH)], idx_v.at[0])
        g = pltpu.async_copy(table_hbm.at[idx_v.at[0]], rows_v0, sem_g0)
        for c in range(1, NCHUNK):
            b = c % 2
            pltpu.sync_copy(
                idx_hbm.at[pl.ds(base + c * CH, CH)], idx_v.at[b]
            )
            if c >= 2:
                pltpu.make_async_copy(
                    rows[b], out_hbm.at[pl.ds(base + (c - 2) * CH, CH)],
                    sem_w[b],
                ).wait()
            g_next = pltpu.async_copy(
                table_hbm.at[idx_v.at[b]], rows[b], sem_g[b]
            )
            g.wait()
            pltpu.async_copy(
                rows[1 - b], out_hbm.at[pl.ds(base + (c - 1) * CH, CH)],
                sem_w[1 - b],
            )
            g = g_next
        last = NCHUNK - 1
        b = last % 2
        g.wait()
        pltpu.async_copy(
            rows[b], out_hbm.at[pl.ds(base + last * CH, CH)], sem_w[b]
        ).wait()
        if NCHUNK >= 2:
            pltpu.make_async_copy(
                rows[1 - b], out_hbm.at[pl.ds(base + (last - 1) * CH, CH)],
                sem_w[1 - b],
            ).wait()

    return _gather


def kernel(x, vectors):
    aux = jnp.concatenate(
        [
            jnp.sum(vectors * vectors, axis=0, keepdims=True),
            jnp.arange(K, dtype=jnp.float32)[None, :],
        ],
        axis=0,
    )
    idx2d, loss_sum = _argmin_call(x, vectors, aux)
    quantized = _make_gather()(vectors.T, idx2d.reshape(-1))
    l = loss_sum[0, 0] / (N * D)
    return quantized, l, l, idx2d
